# baseline (speedup 1.0000x reference)
import jax
import jax.numpy as jnp
from jax import lax
from jax.experimental import pallas as pl
from jax.experimental.pallas import tpu as pltpu
from jax.experimental.pallas import tpu_sc as plsc

N = 100000
C = 64
NC, NS, L = 2, 16, 16
NW = NC * NS
TCOL = 128
NCOLS = N // TCOL
TAIL = N - NCOLS * TCOL
NPAD = (NCOLS + 1) * TCOL
COLS_LO = 25
MAXI = COLS_LO
RING = 4

_mesh = plsc.VectorSubcoreMesh(core_axis_name="c", subcore_axis_name="s")


@jax.jit
def _sc_onehot_t(idx):
    @pl.kernel(
        out_type=jax.ShapeDtypeStruct((C, N), jnp.float32),
        mesh=_mesh,
        scratch_types=[
            pltpu.VMEM((C, TCOL), jnp.float32),
            pltpu.VMEM((C, TCOL), jnp.float32),
            pltpu.VMEM((MAXI * TCOL, ), jnp.int32),
            pltpu.SemaphoreType.DMA,
            pltpu.SemaphoreType.DMA,
            pltpu.SemaphoreType.DMA,
        ],
        compiler_params=pltpu.CompilerParams(
            needs_layout_passes=False,
            use_tc_tiling_on_sc=True,
            disable_bounds_checks=True,
            skip_device_barrier=True,
        ),
    )
    def k(idx_hbm, out_hbm, b0, b1, idx_v, s0, s1, s_idx):
        bufs = (b0, b1)
        sems = (s0, s1)
        wid = lax.axis_index("s") * NC + lax.axis_index("c")
        il = lax.iota(jnp.int32, L)
        ones = jnp.full((L,), 1.0, jnp.float32)
        zeros = jnp.zeros((L,), jnp.float32)

        col0 = jnp.where(wid < 14, wid * 25, wid * 24 + 14)
        n = jnp.where(wid < 14, 25, 24)

        @pl.when(wid < 14)
        def _():
            pltpu.async_copy(
                idx_hbm.at[pl.ds(col0 * TCOL, 25 * TCOL)],
                idx_v.at[pl.ds(0, 25 * TCOL)], s_idx).wait()

        @pl.when((wid >= 14) & (wid < 31))
        def _():
            pltpu.async_copy(
                idx_hbm.at[pl.ds(col0 * TCOL, 24 * TCOL)],
                idx_v.at[pl.ds(0, 24 * TCOL)], s_idx).wait()

        @pl.when(wid == 31)
        def _():
            pltpu.async_copy(
                idx_hbm.at[pl.ds(col0 * TCOL, 23 * TCOL + TAIL)],
                idx_v.at[pl.ds(0, 23 * TCOL + TAIL)], s_idx).wait()

        def zinit(r, carry):
            def zq(q, c2):
                b0[r, pl.ds(q * L, L)] = zeros
                b1[r, pl.ds(q * L, L)] = zeros
                return c2

            return lax.fori_loop(0, TCOL // L, zq, carry)

        lax.fori_loop(0, C, zinit, 0)

        def scatter_col(buf, i, val):
            base = (col0 + i) * TCOL

            def grp(g, carry):
                iv = idx_v[pl.ds(i * TCOL + g * L, L)]
                off = il + g * L
                plsc.store_scatter(buf, [iv, off], val, mask=(off + base) < N)
                return carry

            lax.fori_loop(0, TCOL // L, grp, 0)

        def step(j, carry):
            for half in range(2):
                i = 2 * j + half
                b = bufs[half]
                s = sems[half]

                @pl.when((i >= 2) & (i < n))
                def _():
                    pltpu.make_async_copy(
                        b, out_hbm.at[:, pl.ds((col0 + i - 2) * TCOL, TCOL)],
                        s).wait()
                    scatter_col(b, i - 2, zeros)

                @pl.when(i < n)
                def _():
                    scatter_col(b, i, ones)
                    pltpu.async_copy(
                        b, out_hbm.at[:, pl.ds((col0 + i) * TCOL, TCOL)], s)
            return carry

        lax.fori_loop(0, (MAXI + 1) // 2, step, 0)

        for half in range(2):
            @pl.when(n > half)
            def _():
                last = n - 1 - ((n - 1 - half) % 2)
                pltpu.make_async_copy(
                    bufs[half],
                    out_hbm.at[:, pl.ds((col0 + last) * TCOL, TCOL)],
                    sems[half]).wait()

    return k(idx)


BI = 4096
GRID = (N + BI - 1) // BI


@jax.jit
def _tc_onehot_t(idx):
    def body(idx_ref, o_ref):
        cls = lax.broadcasted_iota(jnp.int32, (C, BI), 0)
        o_ref[...] = (cls == idx_ref[...]).astype(jnp.float32)

    return pl.pallas_call(
        body,
        grid=(GRID,),
        in_specs=[pl.BlockSpec((BI,), lambda b: (b,))],
        out_specs=pl.BlockSpec((C, BI), lambda b: (0, b)),
        out_shape=jax.ShapeDtypeStruct((C, N), jnp.float32),
    )(idx)


def kernel(species_index, pos):
    idx = species_index.astype(jnp.int32)
    o1t = _sc_onehot_t(idx)
    o2t = _tc_onehot_t(idx)
    return (o1t.T.astype(pos.dtype), o2t.T.astype(pos.dtype))


# speedup vs baseline: 1.0854x; 1.0854x over previous
import jax
import jax.numpy as jnp
from jax import lax
from jax.experimental import pallas as pl
from jax.experimental.pallas import tpu as pltpu
from jax.experimental.pallas import tpu_sc as plsc

N = 100000
C = 64
NC, NS, L = 2, 16, 16
NW = NC * NS
TCOL = 128
NCOLS = N // TCOL
TAIL = N - NCOLS * TCOL
MAXI = 25

_mesh = plsc.VectorSubcoreMesh(core_axis_name="c", subcore_axis_name="s")


@jax.jit
def _sc_onehot_t(idx):
    @pl.kernel(
        out_type=jax.ShapeDtypeStruct((C, N), jnp.float32),
        mesh=_mesh,
        scratch_types=[
            pltpu.VMEM((C, TCOL), jnp.float32),
            pltpu.VMEM((C, TCOL), jnp.float32),
            pltpu.VMEM((MAXI * TCOL, ), jnp.int32),
            pltpu.SemaphoreType.DMA,
            pltpu.SemaphoreType.DMA,
            pltpu.SemaphoreType.DMA,
        ],
        compiler_params=pltpu.CompilerParams(
            needs_layout_passes=False,
            use_tc_tiling_on_sc=True,
            disable_bounds_checks=True,
            skip_device_barrier=True,
        ),
    )
    def k(idx_hbm, out_hbm, b0, b1, idx_v, s0, s1, s_idx):
        bufs = (b0, b1)
        sems = (s0, s1)
        wid = lax.axis_index("s") * NC + lax.axis_index("c")
        il = lax.iota(jnp.int32, L)
        ones = jnp.full((L,), 1.0, jnp.float32)
        zeros = jnp.zeros((L,), jnp.float32)

        col0 = jnp.where(wid < 14, wid * 25, wid * 24 + 14)
        n = jnp.where(wid < 14, 25, 24)

        @pl.when(wid < 14)
        def _():
            pltpu.async_copy(
                idx_hbm.at[pl.ds(col0 * TCOL, 25 * TCOL)],
                idx_v.at[pl.ds(0, 25 * TCOL)], s_idx).wait()

        @pl.when((wid >= 14) & (wid < 31))
        def _():
            pltpu.async_copy(
                idx_hbm.at[pl.ds(col0 * TCOL, 24 * TCOL)],
                idx_v.at[pl.ds(0, 24 * TCOL)], s_idx).wait()

        @pl.when(wid == 31)
        def _():
            pltpu.async_copy(
                idx_hbm.at[pl.ds(col0 * TCOL, 23 * TCOL + TAIL)],
                idx_v.at[pl.ds(0, 23 * TCOL + TAIL)], s_idx).wait()

        def zinit(r, carry):
            def zq(q, c2):
                b0[r, pl.ds(q * L, L)] = zeros
                b1[r, pl.ds(q * L, L)] = zeros
                return c2

            return lax.fori_loop(0, TCOL // L, zq, carry)

        lax.fori_loop(0, C, zinit, 0)

        def scatter_col(buf, i, val):
            base = (col0 + i) * TCOL

            def grp(g, carry):
                iv = idx_v[pl.ds(i * TCOL + g * L, L)]
                off = il + g * L
                plsc.store_scatter(buf, [iv, off], val, mask=(off + base) < N)
                return carry

            lax.fori_loop(0, TCOL // L, grp, 0)

        def step(j, carry):
            for half in range(2):
                i = 2 * j + half
                b = bufs[half]
                s = sems[half]

                @pl.when((i >= 2) & (i < n))
                def _():
                    pltpu.make_async_copy(
                        b, out_hbm.at[:, pl.ds((col0 + i - 2) * TCOL, TCOL)],
                        s).wait()
                    scatter_col(b, i - 2, zeros)

                @pl.when(i < n)
                def _():
                    scatter_col(b, i, ones)
                    pltpu.async_copy(
                        b, out_hbm.at[:, pl.ds((col0 + i) * TCOL, TCOL)], s)
            return carry

        lax.fori_loop(0, (MAXI + 1) // 2, step, 0)

        for half in range(2):
            @pl.when(n > half)
            def _():
                last = n - 1 - ((n - 1 - half) % 2)
                pltpu.make_async_copy(
                    bufs[half],
                    out_hbm.at[:, pl.ds((col0 + last) * TCOL, TCOL)],
                    sems[half]).wait()

    return k(idx)


BI = 8192
GRID = (N + BI - 1) // BI


@jax.jit
def _tc_onehot_t(idx):
    def body(idx_ref, o_ref):
        cls = lax.broadcasted_iota(jnp.int32, (C, BI), 0)
        o_ref[...] = (cls == idx_ref[...]).astype(jnp.float32)

    return pl.pallas_call(
        body,
        grid=(GRID,),
        in_specs=[pl.BlockSpec((BI,), lambda b: (b,))],
        out_specs=pl.BlockSpec((C, BI), lambda b: (0, b)),
        out_shape=jax.ShapeDtypeStruct((C, N), jnp.float32),
    )(idx)


def kernel(species_index, pos):
    idx = species_index.astype(jnp.int32)
    o1t = _sc_onehot_t(idx)
    o2t = _tc_onehot_t(idx)
    return (o1t.T.astype(pos.dtype), o2t.T.astype(pos.dtype))


# speedup vs baseline: 1.1127x; 1.0251x over previous
import jax
import jax.numpy as jnp
from jax import lax
from jax.experimental import pallas as pl
from jax.experimental.pallas import tpu as pltpu
from jax.experimental.pallas import tpu_sc as plsc

N = 100000
C = 64
NC, NS, L = 2, 16, 16
NW = NC * NS
TCOL = 128
NCOLS = N // TCOL
TAIL = N - NCOLS * TCOL
MAXI = 25

_mesh = plsc.VectorSubcoreMesh(core_axis_name="c", subcore_axis_name="s")


@jax.jit
def _sc_onehot_t(idx):
    @pl.kernel(
        out_type=jax.ShapeDtypeStruct((C, N), jnp.float32),
        mesh=_mesh,
        scratch_types=[
            pltpu.VMEM((C, TCOL), jnp.float32),
            pltpu.VMEM((C, TCOL), jnp.float32),
            pltpu.VMEM((MAXI * TCOL, ), jnp.int32),
            pltpu.SemaphoreType.DMA,
            pltpu.SemaphoreType.DMA,
            pltpu.SemaphoreType.DMA,
        ],
        compiler_params=pltpu.CompilerParams(
            needs_layout_passes=False,
            use_tc_tiling_on_sc=True,
            disable_bounds_checks=True,
            skip_device_barrier=True,
        ),
    )
    def k(idx_hbm, out_hbm, b0, b1, idx_v, s0, s1, s_idx):
        bufs = (b0, b1)
        sems = (s0, s1)
        wid = lax.axis_index("s") * NC + lax.axis_index("c")
        il = lax.iota(jnp.int32, L)
        ones = jnp.full((L,), 1.0, jnp.float32)
        zeros = jnp.zeros((L,), jnp.float32)

        col0 = jnp.where(wid < 14, wid * 25, wid * 24 + 14)
        n = jnp.where(wid < 14, 25, 24)

        @pl.when(wid < 14)
        def _():
            pltpu.async_copy(
                idx_hbm.at[pl.ds(col0 * TCOL, 25 * TCOL)],
                idx_v.at[pl.ds(0, 25 * TCOL)], s_idx).wait()

        @pl.when((wid >= 14) & (wid < 31))
        def _():
            pltpu.async_copy(
                idx_hbm.at[pl.ds(col0 * TCOL, 24 * TCOL)],
                idx_v.at[pl.ds(0, 24 * TCOL)], s_idx).wait()

        @pl.when(wid == 31)
        def _():
            pltpu.async_copy(
                idx_hbm.at[pl.ds(col0 * TCOL, 23 * TCOL + TAIL)],
                idx_v.at[pl.ds(0, 23 * TCOL + TAIL)], s_idx).wait()

        def zinit(r, carry):
            def zq(q, c2):
                b0[r, pl.ds(q * L, L)] = zeros
                b1[r, pl.ds(q * L, L)] = zeros
                return c2

            return lax.fori_loop(0, TCOL // L, zq, carry)

        lax.fori_loop(0, C, zinit, 0)

        def scatter_col(buf, i, val):
            base = (col0 + i) * TCOL

            def grp(g, carry):
                iv = idx_v[pl.ds(i * TCOL + g * L, L)]
                off = il + g * L
                plsc.store_scatter(buf, [iv, off], val, mask=(off + base) < N)
                return carry

            lax.fori_loop(0, TCOL // L, grp, 0)

        def step(j, carry):
            for half in range(2):
                i = 2 * j + half
                b = bufs[half]
                s = sems[half]

                @pl.when((i >= 2) & (i < n))
                def _():
                    pltpu.make_async_copy(
                        b, out_hbm.at[:, pl.ds((col0 + i - 2) * TCOL, TCOL)],
                        s).wait()
                    scatter_col(b, i - 2, zeros)

                @pl.when(i < n)
                def _():
                    scatter_col(b, i, ones)
                    pltpu.async_copy(
                        b, out_hbm.at[:, pl.ds((col0 + i) * TCOL, TCOL)], s)
            return carry

        lax.fori_loop(0, (MAXI + 1) // 2, step, 0)

        for half in range(2):
            @pl.when(n > half)
            def _():
                last = n - 1 - ((n - 1 - half) % 2)
                pltpu.make_async_copy(
                    bufs[half],
                    out_hbm.at[:, pl.ds((col0 + last) * TCOL, TCOL)],
                    sems[half]).wait()

    return k(idx)


BI = 16384
GRID = (N + BI - 1) // BI


@jax.jit
def _tc_onehot_t(idx):
    def body(idx_ref, o_ref):
        cls = lax.broadcasted_iota(jnp.int32, (C, BI), 0)
        o_ref[...] = (cls == idx_ref[...]).astype(jnp.float32)

    return pl.pallas_call(
        body,
        grid=(GRID,),
        in_specs=[pl.BlockSpec((BI,), lambda b: (b,))],
        out_specs=pl.BlockSpec((C, BI), lambda b: (0, b)),
        out_shape=jax.ShapeDtypeStruct((C, N), jnp.float32),
    )(idx)


def kernel(species_index, pos):
    idx = species_index.astype(jnp.int32)
    o1t = _sc_onehot_t(idx)
    o2t = _tc_onehot_t(idx)
    return (o1t.T.astype(pos.dtype), o2t.T.astype(pos.dtype))


# speedup vs baseline: 1.1155x; 1.0026x over previous
import jax
import jax.numpy as jnp
from jax import lax
from jax.experimental import pallas as pl
from jax.experimental.pallas import tpu as pltpu
from jax.experimental.pallas import tpu_sc as plsc

N = 100000
C = 64
NC, NS, L = 2, 16, 16
NW = NC * NS
TCOL = 128
NCOLS = N // TCOL
TAIL = N - NCOLS * TCOL
MAXI = 25

_mesh = plsc.VectorSubcoreMesh(core_axis_name="c", subcore_axis_name="s")


@jax.jit
def _sc_onehot_t(idx):
    @pl.kernel(
        out_type=jax.ShapeDtypeStruct((C, N), jnp.float32),
        mesh=_mesh,
        scratch_types=[
            pltpu.VMEM((C, TCOL), jnp.float32),
            pltpu.VMEM((C, TCOL), jnp.float32),
            pltpu.VMEM((MAXI * TCOL, ), jnp.int32),
            pltpu.SemaphoreType.DMA,
            pltpu.SemaphoreType.DMA,
            pltpu.SemaphoreType.DMA,
        ],
        compiler_params=pltpu.CompilerParams(
            needs_layout_passes=False,
            use_tc_tiling_on_sc=True,
            disable_bounds_checks=True,
            skip_device_barrier=True,
        ),
    )
    def k(idx_hbm, out_hbm, b0, b1, idx_v, s0, s1, s_idx):
        bufs = (b0, b1)
        sems = (s0, s1)
        wid = lax.axis_index("s") * NC + lax.axis_index("c")
        il = lax.iota(jnp.int32, L)
        ones = jnp.full((L,), 1.0, jnp.float32)
        zeros = jnp.zeros((L,), jnp.float32)

        col0 = jnp.where(wid < 14, wid * 25, wid * 24 + 14)
        n = jnp.where(wid < 14, 25, 24)

        @pl.when(wid < 14)
        def _():
            pltpu.async_copy(
                idx_hbm.at[pl.ds(col0 * TCOL, 25 * TCOL)],
                idx_v.at[pl.ds(0, 25 * TCOL)], s_idx).wait()

        @pl.when((wid >= 14) & (wid < 31))
        def _():
            pltpu.async_copy(
                idx_hbm.at[pl.ds(col0 * TCOL, 24 * TCOL)],
                idx_v.at[pl.ds(0, 24 * TCOL)], s_idx).wait()

        @pl.when(wid == 31)
        def _():
            pltpu.async_copy(
                idx_hbm.at[pl.ds(col0 * TCOL, 23 * TCOL + TAIL)],
                idx_v.at[pl.ds(0, 23 * TCOL + TAIL)], s_idx).wait()

        def zinit(r, carry):
            def zq(q, c2):
                b0[r, pl.ds(q * L, L)] = zeros
                b1[r, pl.ds(q * L, L)] = zeros
                return c2

            return lax.fori_loop(0, TCOL // L, zq, carry)

        lax.fori_loop(0, C, zinit, 0)

        def scatter_col(buf, i, val):
            base = (col0 + i) * TCOL

            def grp(g, carry):
                iv = idx_v[pl.ds(i * TCOL + g * L, L)]
                off = il + g * L
                plsc.store_scatter(buf, [iv, off], val, mask=(off + base) < N)
                return carry

            lax.fori_loop(0, TCOL // L, grp, 0)

        def step(j, carry):
            for half in range(2):
                i = 2 * j + half
                b = bufs[half]
                s = sems[half]

                @pl.when((i >= 2) & (i < n))
                def _():
                    pltpu.make_async_copy(
                        b, out_hbm.at[:, pl.ds((col0 + i - 2) * TCOL, TCOL)],
                        s).wait()
                    scatter_col(b, i - 2, zeros)

                @pl.when(i < n)
                def _():
                    scatter_col(b, i, ones)
                    pltpu.async_copy(
                        b, out_hbm.at[:, pl.ds((col0 + i) * TCOL, TCOL)], s)
            return carry

        lax.fori_loop(0, (MAXI + 1) // 2, step, 0)

        for half in range(2):
            @pl.when(n > half)
            def _():
                last = n - 1 - ((n - 1 - half) % 2)
                pltpu.make_async_copy(
                    bufs[half],
                    out_hbm.at[:, pl.ds((col0 + last) * TCOL, TCOL)],
                    sems[half]).wait()

    return k(idx)


BI = 32768
GRID = (N + BI - 1) // BI


@jax.jit
def _tc_onehot_t(idx):
    def body(idx_ref, o_ref):
        cls = lax.broadcasted_iota(jnp.int32, (C, BI), 0)
        o_ref[...] = (cls == idx_ref[...]).astype(jnp.float32)

    return pl.pallas_call(
        body,
        grid=(GRID,),
        in_specs=[pl.BlockSpec((BI,), lambda b: (b,))],
        out_specs=pl.BlockSpec((C, BI), lambda b: (0, b)),
        out_shape=jax.ShapeDtypeStruct((C, N), jnp.float32),
    )(idx)


def kernel(species_index, pos):
    idx = species_index.astype(jnp.int32)
    o1t = _sc_onehot_t(idx)
    o2t = _tc_onehot_t(idx)
    return (o1t.T.astype(pos.dtype), o2t.T.astype(pos.dtype))
